# baseline (speedup 1.0000x reference)
import functools

import jax
import jax.numpy as jnp
from jax import lax
from jax.experimental import pallas as pl
from jax.experimental.pallas import tpu as pltpu
from jax.experimental.pallas import tpu_sc as plsc

NUM_FRAME = 8
TOPK = 64
TOP_ATTN = 6
P = 196
D = 768
NUM_LAYERS = 12
NUM_HEADS = 12
SEQ = P + 1
W = 208

SC_CORES = 2
SC_SUBCORES = 16
SC_WORKERS = SC_CORES * SC_SUBCORES

NL = NUM_LAYERS - TOP_ATTN
HG = 2
HPG = NUM_HEADS // HG

_OFFS = [k * 16 for k in range(SEQ // 16)] + [SEQ - 16]


@functools.lru_cache(maxsize=None)
def _make_sc_scores(batch):
    n_units = batch * NL * HG
    assert n_units <= SC_WORKERS
    mesh = plsc.VectorSubcoreMesh(core_axis_name="c", subcore_axis_name="s")

    @functools.partial(
        pl.kernel,
        mesh=mesh,
        compiler_params=pltpu.CompilerParams(use_tc_tiling_on_sc=True),
        out_type=jax.ShapeDtypeStruct((n_units, NUM_FRAME, W), jnp.float32),
        scratch_types=[
            pltpu.VMEM((HPG, NUM_FRAME, SEQ), jnp.float32),
            pltpu.VMEM((NUM_FRAME, W), jnp.float32),
            pltpu.SemaphoreType.DMA,
            pltpu.SemaphoreType.DMA,
        ],
    )
    def sc_scores(attn_hbm, out_hbm, buf, acc, sem, sem2):
        wid = lax.axis_index("s") * SC_CORES + lax.axis_index("c")

        @pl.when(wid < n_units)
        def _():
            b = wid // (NL * HG)
            rem = wid % (NL * HG)
            l = TOP_ATTN + rem // HG
            hg = rem % HG
            hh = HPG // 2
            h1 = pltpu.async_copy(
                attn_hbm.at[b, l, pl.ds(hg * HPG, hh), 0, :, :],
                buf.at[pl.ds(0, hh)],
                sem,
            )
            h2 = pltpu.async_copy(
                attn_hbm.at[b, l, pl.ds(hg * HPG + hh, hh), 0, :, :],
                buf.at[pl.ds(hh, hh)],
                sem2,
            )
            h1.wait()
            for t in range(NUM_FRAME):
                for o in _OFFS:
                    s = buf[0, t, pl.ds(o, 16)]
                    for j in range(1, hh):
                        s = s + buf[j, t, pl.ds(o, 16)]
                    acc[t, pl.ds(o, 16)] = s
            h2.wait()
            for t in range(NUM_FRAME):
                for o in _OFFS:
                    s = acc[t, pl.ds(o, 16)]
                    for j in range(hh, HPG):
                        s = s + buf[j, t, pl.ds(o, 16)]
                    acc[t, pl.ds(o, 16)] = s
            pltpu.sync_copy(acc, out_hbm.at[wid])

    return sc_scores


def _topk_body(s_ref, idx_ref, gidx_ref, *, batch):
    rows = batch * NUM_FRAME
    s = jnp.sum(s_ref[...], axis=1).reshape(rows, W)

    lane = lax.broadcasted_iota(jnp.int32, (rows, W), 1)
    valid = (lane >= 1) & (lane < SEQ)
    s = jnp.where(valid, s, -jnp.inf)

    sj3 = s[:, None, :]
    jl = lax.broadcasted_iota(jnp.int32, (rows, 16, W), 2)
    il0 = lax.broadcasted_iota(jnp.int32, (rows, 16, W), 1)
    rank_chunks = []
    for ic in range(W // 16):
        si3 = s[:, ic * 16:(ic + 1) * 16, None]
        beats = (sj3 > si3) | ((sj3 == si3) & (jl < il0 + ic * 16))
        cnt = jnp.sum(jnp.where(beats, 1.0, 0.0), axis=2)
        rank_chunks.append(cnt)
    rank = jnp.concatenate(rank_chunks, axis=1).astype(jnp.int32)

    pp = lax.broadcasted_iota(jnp.int32, (rows, TOPK, W), 1)
    sel = rank[:, None, :] == pp
    lane3 = lax.broadcasted_iota(jnp.int32, (rows, TOPK, W), 2)
    idx = jnp.sum(jnp.where(sel, lane3 - 1, 0), axis=2)

    idx_ref[...] = idx
    row = lax.broadcasted_iota(jnp.int32, (rows, TOPK), 0)
    gidx = idx + row * P
    gidx_ref[...] = jnp.concatenate([gidx, gidx], axis=1)


@functools.lru_cache(maxsize=None)
def _make_sc_gather(rows, d):
    n_rows = rows * TOPK
    per_w = n_rows // SC_WORKERS
    mesh = plsc.VectorSubcoreMesh(core_axis_name="c", subcore_axis_name="s")

    @functools.partial(
        pl.kernel,
        mesh=mesh,
        out_type=jax.ShapeDtypeStruct((n_rows, d), jnp.float32),
        scratch_types=[
            pltpu.VMEM((per_w,), jnp.int32),
            pltpu.VMEM((per_w, d), jnp.float32),
            pltpu.SemaphoreType.DMA,
        ],
    )
    def sc_gather(table_hbm, gidx_hbm, out_hbm, idx_v, rows_v, sem):
        wid = lax.axis_index("s") * SC_CORES + lax.axis_index("c")
        r = wid // 2
        c = wid % 2
        pltpu.sync_copy(gidx_hbm.at[r, pl.ds(c * per_w, per_w)], idx_v)
        pltpu.async_copy(table_hbm.at[idx_v], rows_v, sem).wait()
        pltpu.sync_copy(rows_v, out_hbm.at[pl.ds(r * TOPK + c * per_w, per_w)])

    return sc_gather


def kernel(tokens, attn_maps):
    B = tokens.shape[0]
    rows = B * NUM_FRAME

    am_t = jnp.transpose(attn_maps, (0, 2, 3, 4, 1, 5))
    parts = _make_sc_scores(B)(am_t)

    idx, gidx = pl.pallas_call(
        functools.partial(_topk_body, batch=B),
        grid=(1,),
        in_specs=[
            pl.BlockSpec((B, NL * HG, NUM_FRAME, W), lambda i: (0, 0, 0, 0))
        ],
        out_specs=[
            pl.BlockSpec((rows, TOPK), lambda i: (0, 0)),
            pl.BlockSpec((rows, 2 * TOPK), lambda i: (0, 0)),
        ],
        out_shape=[
            jax.ShapeDtypeStruct((rows, TOPK), jnp.int32),
            jax.ShapeDtypeStruct((rows, 2 * TOPK), jnp.int32),
        ],
    )(parts.reshape(B, NL * HG, NUM_FRAME, W))

    gather = _make_sc_gather(rows, D)
    out = gather(tokens.reshape(B * NUM_FRAME * P, D), gidx)

    return out.reshape(B, NUM_FRAME * TOPK, D), idx.reshape(B, NUM_FRAME, TOPK)


# speedup vs baseline: 1.0105x; 1.0105x over previous
import functools

import jax
import jax.numpy as jnp
from jax import lax
from jax.experimental import pallas as pl
from jax.experimental.pallas import tpu as pltpu
from jax.experimental.pallas import tpu_sc as plsc

NUM_FRAME = 8
TOPK = 64
TOP_ATTN = 6
P = 196
D = 768
NUM_LAYERS = 12
NUM_HEADS = 12
SEQ = P + 1
W = 208

SC_CORES = 2
SC_SUBCORES = 16
SC_WORKERS = SC_CORES * SC_SUBCORES

NL = NUM_LAYERS - TOP_ATTN
HG = 2
HPG = NUM_HEADS // HG

_OFFS = [k * 16 for k in range(SEQ // 16)] + [SEQ - 16]


@functools.lru_cache(maxsize=None)
def _make_sc_scores(batch):
    n_units = batch * NL * HG
    assert n_units <= SC_WORKERS
    mesh = plsc.VectorSubcoreMesh(core_axis_name="c", subcore_axis_name="s")

    @functools.partial(
        pl.kernel,
        mesh=mesh,
        compiler_params=pltpu.CompilerParams(use_tc_tiling_on_sc=True),
        out_type=jax.ShapeDtypeStruct((n_units, NUM_FRAME, W), jnp.float32),
        scratch_types=[
            pltpu.VMEM((HPG, NUM_FRAME, SEQ), jnp.float32),
            pltpu.VMEM((NUM_FRAME, W), jnp.float32),
            pltpu.SemaphoreType.DMA,
        ],
    )
    def sc_scores(attn_hbm, out_hbm, buf, acc, sem):
        wid = lax.axis_index("s") * SC_CORES + lax.axis_index("c")

        @pl.when(wid < n_units)
        def _():
            b = wid // (NL * HG)
            rem = wid % (NL * HG)
            l = TOP_ATTN + rem // HG
            hg = rem % HG
            pltpu.async_copy(
                attn_hbm.at[b, l, pl.ds(hg * HPG, HPG), 0, :, :],
                buf,
                sem,
            ).wait()
            for t in range(NUM_FRAME):
                for o in _OFFS:
                    s = buf[0, t, pl.ds(o, 16)]
                    for j in range(1, HPG):
                        s = s + buf[j, t, pl.ds(o, 16)]
                    acc[t, pl.ds(o, 16)] = s
            pltpu.sync_copy(acc, out_hbm.at[wid])

    return sc_scores


def _topk_body(s_ref, idx_ref, gidx_ref, *, batch):
    rows = batch * NUM_FRAME
    s = jnp.sum(s_ref[...], axis=1).reshape(rows, W)

    lane = lax.broadcasted_iota(jnp.int32, (rows, W), 1)
    valid = (lane >= 1) & (lane < SEQ)
    s = jnp.where(valid, s, -jnp.inf)

    sj3 = s[:, None, :]
    jl = lax.broadcasted_iota(jnp.int32, (rows, 16, W), 2)
    il0 = lax.broadcasted_iota(jnp.int32, (rows, 16, W), 1)
    rank_chunks = []
    for ic in range(W // 16):
        si3 = s[:, ic * 16:(ic + 1) * 16, None]
        beats = (sj3 > si3) | ((sj3 == si3) & (jl < il0 + ic * 16))
        cnt = jnp.sum(jnp.where(beats, 1.0, 0.0), axis=2)
        rank_chunks.append(cnt)
    rank = jnp.concatenate(rank_chunks, axis=1).astype(jnp.int32)

    pp = lax.broadcasted_iota(jnp.int32, (rows, TOPK, W), 1)
    sel = rank[:, None, :] == pp
    lane3 = lax.broadcasted_iota(jnp.int32, (rows, TOPK, W), 2)
    idx = jnp.sum(jnp.where(sel, lane3 - 1, 0), axis=2)

    idx_ref[...] = idx
    row = lax.broadcasted_iota(jnp.int32, (rows, TOPK), 0)
    gidx = idx + row * P
    gidx_ref[...] = jnp.concatenate([gidx, gidx], axis=1)


@functools.lru_cache(maxsize=None)
def _make_sc_gather(rows, d):
    n_rows = rows * TOPK
    per_w = n_rows // SC_WORKERS
    mesh = plsc.VectorSubcoreMesh(core_axis_name="c", subcore_axis_name="s")

    @functools.partial(
        pl.kernel,
        mesh=mesh,
        out_type=jax.ShapeDtypeStruct((n_rows, d), jnp.float32),
        scratch_types=[
            pltpu.VMEM((per_w,), jnp.int32),
            pltpu.VMEM((per_w, d), jnp.float32),
            pltpu.SemaphoreType.DMA,
        ],
    )
    def sc_gather(table_hbm, gidx_hbm, out_hbm, idx_v, rows_v, sem):
        wid = lax.axis_index("s") * SC_CORES + lax.axis_index("c")
        r = wid // 2
        c = wid % 2
        pltpu.sync_copy(gidx_hbm.at[r, pl.ds(c * per_w, per_w)], idx_v)
        pltpu.async_copy(table_hbm.at[idx_v], rows_v, sem).wait()
        pltpu.sync_copy(rows_v, out_hbm.at[pl.ds(r * TOPK + c * per_w, per_w)])

    return sc_gather


def kernel(tokens, attn_maps):
    B = tokens.shape[0]
    rows = B * NUM_FRAME

    am_t = jnp.transpose(attn_maps, (0, 2, 3, 4, 1, 5))
    parts = _make_sc_scores(B)(am_t)

    idx, gidx = pl.pallas_call(
        functools.partial(_topk_body, batch=B),
        grid=(1,),
        in_specs=[
            pl.BlockSpec((B, NL * HG, NUM_FRAME, W), lambda i: (0, 0, 0, 0))
        ],
        out_specs=[
            pl.BlockSpec((rows, TOPK), lambda i: (0, 0)),
            pl.BlockSpec((rows, 2 * TOPK), lambda i: (0, 0)),
        ],
        out_shape=[
            jax.ShapeDtypeStruct((rows, TOPK), jnp.int32),
            jax.ShapeDtypeStruct((rows, 2 * TOPK), jnp.int32),
        ],
    )(parts.reshape(B, NL * HG, NUM_FRAME, W))

    gather = _make_sc_gather(rows, D)
    out = gather(tokens.reshape(B * NUM_FRAME * P, D), gidx)

    return out.reshape(B, NUM_FRAME * TOPK, D), idx.reshape(B, NUM_FRAME, TOPK)


# speedup vs baseline: 1.0126x; 1.0021x over previous
import functools

import jax
import jax.numpy as jnp
from jax import lax
from jax.experimental import pallas as pl
from jax.experimental.pallas import tpu as pltpu
from jax.experimental.pallas import tpu_sc as plsc

NUM_FRAME = 8
TOPK = 64
TOP_ATTN = 6
P = 196
D = 768
NUM_LAYERS = 12
NUM_HEADS = 12
SEQ = P + 1
W = 208

SC_CORES = 2
SC_SUBCORES = 16
SC_WORKERS = SC_CORES * SC_SUBCORES

NL = NUM_LAYERS - TOP_ATTN
HG = 2
HPG = NUM_HEADS // HG

_OFFS = [k * 16 for k in range(SEQ // 16)] + [SEQ - 16]


@functools.lru_cache(maxsize=None)
def _make_sc_scores(batch):
    n_units = batch * NL * HG
    assert n_units <= SC_WORKERS
    mesh = plsc.VectorSubcoreMesh(core_axis_name="c", subcore_axis_name="s")

    @functools.partial(
        pl.kernel,
        mesh=mesh,
        compiler_params=pltpu.CompilerParams(use_tc_tiling_on_sc=True),
        out_type=jax.ShapeDtypeStruct((n_units, NUM_FRAME, W), jnp.float32),
        scratch_types=[
            pltpu.VMEM((HPG // 2, NUM_FRAME, SEQ), jnp.float32),
            pltpu.VMEM((HPG // 2, NUM_FRAME, SEQ), jnp.float32),
            pltpu.VMEM((NUM_FRAME, W), jnp.float32),
            pltpu.SemaphoreType.DMA,
            pltpu.SemaphoreType.DMA,
        ],
    )
    def sc_scores(attn_hbm, out_hbm, buf_a, buf_b, acc, sem, sem2):
        wid = lax.axis_index("s") * SC_CORES + lax.axis_index("c")

        @pl.when(wid < n_units)
        def _():
            b = wid // (NL * HG)
            rem = wid % (NL * HG)
            l = TOP_ATTN + rem // HG
            hg = rem % HG
            hh = HPG // 2
            h1 = pltpu.async_copy(
                attn_hbm.at[b, l, pl.ds(hg * HPG, hh), 0, :, :], buf_a, sem
            )
            h2 = pltpu.async_copy(
                attn_hbm.at[b, l, pl.ds(hg * HPG + hh, hh), 0, :, :], buf_b, sem
            )
            h1.wait()
            h2.wait()
            for t in range(NUM_FRAME):
                for o in _OFFS:
                    s = buf_a[0, t, pl.ds(o, 16)]
                    for j in range(1, hh):
                        s = s + buf_a[j, t, pl.ds(o, 16)]
                    for j in range(hh):
                        s = s + buf_b[j, t, pl.ds(o, 16)]
                    acc[t, pl.ds(o, 16)] = s
            pltpu.sync_copy(acc, out_hbm.at[wid])

    return sc_scores


def _topk_body(s_ref, idx_ref, gidx_ref, *, batch):
    rows = batch * NUM_FRAME
    s = jnp.sum(s_ref[...], axis=1).reshape(rows, W)

    lane = lax.broadcasted_iota(jnp.int32, (rows, W), 1)
    valid = (lane >= 1) & (lane < SEQ)
    s = jnp.where(valid, s, -jnp.inf)

    sj3 = s[:, None, :]
    jl = lax.broadcasted_iota(jnp.int32, (rows, 16, W), 2)
    il0 = lax.broadcasted_iota(jnp.int32, (rows, 16, W), 1)
    rank_chunks = []
    for ic in range(W // 16):
        si3 = s[:, ic * 16:(ic + 1) * 16, None]
        beats = (sj3 > si3) | ((sj3 == si3) & (jl < il0 + ic * 16))
        cnt = jnp.sum(jnp.where(beats, 1.0, 0.0), axis=2)
        rank_chunks.append(cnt)
    rank = jnp.concatenate(rank_chunks, axis=1).astype(jnp.int32)

    pp = lax.broadcasted_iota(jnp.int32, (rows, TOPK, W), 1)
    sel = rank[:, None, :] == pp
    lane3 = lax.broadcasted_iota(jnp.int32, (rows, TOPK, W), 2)
    idx = jnp.sum(jnp.where(sel, lane3 - 1, 0), axis=2)

    idx_ref[...] = idx
    row = lax.broadcasted_iota(jnp.int32, (rows, TOPK), 0)
    gidx = idx + row * P
    gidx_ref[...] = jnp.concatenate([gidx, gidx], axis=1)


@functools.lru_cache(maxsize=None)
def _make_sc_gather(rows, d):
    n_rows = rows * TOPK
    per_w = n_rows // SC_WORKERS
    mesh = plsc.VectorSubcoreMesh(core_axis_name="c", subcore_axis_name="s")

    @functools.partial(
        pl.kernel,
        mesh=mesh,
        out_type=jax.ShapeDtypeStruct((n_rows, d), jnp.float32),
        scratch_types=[
            pltpu.VMEM((per_w,), jnp.int32),
            pltpu.VMEM((per_w, d), jnp.float32),
            pltpu.SemaphoreType.DMA,
        ],
    )
    def sc_gather(table_hbm, gidx_hbm, out_hbm, idx_v, rows_v, sem):
        wid = lax.axis_index("s") * SC_CORES + lax.axis_index("c")
        r = wid // 2
        c = wid % 2
        pltpu.sync_copy(gidx_hbm.at[r, pl.ds(c * per_w, per_w)], idx_v)
        pltpu.async_copy(table_hbm.at[idx_v], rows_v, sem).wait()
        pltpu.sync_copy(rows_v, out_hbm.at[pl.ds(r * TOPK + c * per_w, per_w)])

    return sc_gather


def kernel(tokens, attn_maps):
    B = tokens.shape[0]
    rows = B * NUM_FRAME

    am_t = jnp.transpose(attn_maps, (0, 2, 3, 4, 1, 5))
    parts = _make_sc_scores(B)(am_t)

    idx, gidx = pl.pallas_call(
        functools.partial(_topk_body, batch=B),
        grid=(1,),
        in_specs=[
            pl.BlockSpec((B, NL * HG, NUM_FRAME, W), lambda i: (0, 0, 0, 0))
        ],
        out_specs=[
            pl.BlockSpec((rows, TOPK), lambda i: (0, 0)),
            pl.BlockSpec((rows, 2 * TOPK), lambda i: (0, 0)),
        ],
        out_shape=[
            jax.ShapeDtypeStruct((rows, TOPK), jnp.int32),
            jax.ShapeDtypeStruct((rows, 2 * TOPK), jnp.int32),
        ],
    )(parts.reshape(B, NL * HG, NUM_FRAME, W))

    gather = _make_sc_gather(rows, D)
    out = gather(tokens.reshape(B * NUM_FRAME * P, D), gidx)

    return out.reshape(B, NUM_FRAME * TOPK, D), idx.reshape(B, NUM_FRAME, TOPK)


# speedup vs baseline: 1.0155x; 1.0028x over previous
import functools

import jax
import jax.numpy as jnp
from jax import lax
from jax.experimental import pallas as pl
from jax.experimental.pallas import tpu as pltpu
from jax.experimental.pallas import tpu_sc as plsc

NUM_FRAME = 8
TOPK = 64
TOP_ATTN = 6
P = 196
D = 768
NUM_LAYERS = 12
NUM_HEADS = 12
SEQ = P + 1
W = 208

SC_CORES = 2
SC_SUBCORES = 16
SC_WORKERS = SC_CORES * SC_SUBCORES

NL = NUM_LAYERS - TOP_ATTN
HG = 2
HPG = NUM_HEADS // HG

_OFFS = [k * 16 for k in range(SEQ // 16)] + [SEQ - 16]


@functools.lru_cache(maxsize=None)
def _make_sc_scores(batch):
    n_units = batch * NL * HG
    assert n_units <= SC_WORKERS
    mesh = plsc.VectorSubcoreMesh(core_axis_name="c", subcore_axis_name="s")

    @functools.partial(
        pl.kernel,
        mesh=mesh,
        compiler_params=pltpu.CompilerParams(use_tc_tiling_on_sc=True),
        out_type=jax.ShapeDtypeStruct((n_units, NUM_FRAME, W), jnp.float32),
        scratch_types=[
            pltpu.VMEM((HPG // 2, NUM_FRAME, SEQ), jnp.float32),
            pltpu.VMEM((HPG // 2, NUM_FRAME, SEQ), jnp.float32),
            pltpu.VMEM((NUM_FRAME, W), jnp.float32),
            pltpu.SemaphoreType.DMA,
        ],
    )
    def sc_scores(attn_hbm, out_hbm, buf_a, buf_b, acc, sem):
        wid = lax.axis_index("s") * SC_CORES + lax.axis_index("c")

        @pl.when(wid < n_units)
        def _():
            b = wid // (NL * HG)
            rem = wid % (NL * HG)
            l = TOP_ATTN + rem // HG
            hg = rem % HG
            hh = HPG // 2
            h1 = pltpu.async_copy(
                attn_hbm.at[b, l, pl.ds(hg * HPG, hh), 0, :, :], buf_a, sem
            )
            h2 = pltpu.async_copy(
                attn_hbm.at[b, l, pl.ds(hg * HPG + hh, hh), 0, :, :], buf_b, sem
            )
            h1.wait()
            h2.wait()
            for t in range(NUM_FRAME):
                for o in _OFFS:
                    s = buf_a[0, t, pl.ds(o, 16)]
                    for j in range(1, hh):
                        s = s + buf_a[j, t, pl.ds(o, 16)]
                    for j in range(hh):
                        s = s + buf_b[j, t, pl.ds(o, 16)]
                    acc[t, pl.ds(o, 16)] = s
            pltpu.sync_copy(acc, out_hbm.at[wid])

    return sc_scores


def _topk_body(s_ref, idx_ref, gidx_ref, *, batch):
    rows = batch * NUM_FRAME
    s = jnp.sum(s_ref[...], axis=1).reshape(rows, W)

    lane = lax.broadcasted_iota(jnp.int32, (rows, W), 1)
    valid = (lane >= 1) & (lane < SEQ)
    s = jnp.where(valid, s, -jnp.inf)

    sj3 = s[:, None, :]
    jl = lax.broadcasted_iota(jnp.int32, (rows, 16, W), 2)
    il0 = lax.broadcasted_iota(jnp.int32, (rows, 16, W), 1)
    rank_chunks = []
    for ic in range(W // 16):
        si3 = s[:, ic * 16:(ic + 1) * 16, None]
        beats = (sj3 > si3) | ((sj3 == si3) & (jl < il0 + ic * 16))
        cnt = jnp.sum(jnp.where(beats, 1.0, 0.0), axis=2)
        rank_chunks.append(cnt)
    rank = jnp.concatenate(rank_chunks, axis=1).astype(jnp.int32)

    pp = lax.broadcasted_iota(jnp.int32, (rows, TOPK, W), 1)
    sel = rank[:, None, :] == pp
    lane3 = lax.broadcasted_iota(jnp.int32, (rows, TOPK, W), 2)
    idx = jnp.sum(jnp.where(sel, lane3 - 1, 0), axis=2)

    idx_ref[...] = idx
    row = lax.broadcasted_iota(jnp.int32, (rows, TOPK), 0)
    gidx = idx + row * P
    gidx_ref[...] = jnp.concatenate([gidx, gidx], axis=1)


@functools.lru_cache(maxsize=None)
def _make_sc_gather(rows, d):
    n_rows = rows * TOPK
    per_w = n_rows // SC_WORKERS
    mesh = plsc.VectorSubcoreMesh(core_axis_name="c", subcore_axis_name="s")

    @functools.partial(
        pl.kernel,
        mesh=mesh,
        out_type=jax.ShapeDtypeStruct((n_rows, d), jnp.float32),
        scratch_types=[
            pltpu.VMEM((per_w,), jnp.int32),
            pltpu.VMEM((per_w, d), jnp.float32),
            pltpu.SemaphoreType.DMA,
        ],
    )
    def sc_gather(table_hbm, gidx_hbm, out_hbm, idx_v, rows_v, sem):
        wid = lax.axis_index("s") * SC_CORES + lax.axis_index("c")
        r = wid // 2
        c = wid % 2
        pltpu.sync_copy(gidx_hbm.at[r, pl.ds(c * per_w, per_w)], idx_v)
        pltpu.async_copy(table_hbm.at[idx_v], rows_v, sem).wait()
        pltpu.sync_copy(rows_v, out_hbm.at[pl.ds(r * TOPK + c * per_w, per_w)])

    return sc_gather


def kernel(tokens, attn_maps):
    B = tokens.shape[0]
    rows = B * NUM_FRAME

    am_t = jnp.transpose(attn_maps, (0, 2, 3, 4, 1, 5))
    parts = _make_sc_scores(B)(am_t)

    idx, gidx = pl.pallas_call(
        functools.partial(_topk_body, batch=B),
        grid=(1,),
        in_specs=[
            pl.BlockSpec((B, NL * HG, NUM_FRAME, W), lambda i: (0, 0, 0, 0))
        ],
        out_specs=[
            pl.BlockSpec((rows, TOPK), lambda i: (0, 0)),
            pl.BlockSpec((rows, 2 * TOPK), lambda i: (0, 0)),
        ],
        out_shape=[
            jax.ShapeDtypeStruct((rows, TOPK), jnp.int32),
            jax.ShapeDtypeStruct((rows, 2 * TOPK), jnp.int32),
        ],
    )(parts.reshape(B, NL * HG, NUM_FRAME, W))

    gather = _make_sc_gather(rows, D)
    out = gather(tokens.reshape(B * NUM_FRAME * P, D), gidx)

    return out.reshape(B, NUM_FRAME * TOPK, D), idx.reshape(B, NUM_FRAME, TOPK)
